# baseline (speedup 1.0000x reference)
import jax
import jax.numpy as jnp
from jax.experimental import pallas as pl
from jax.experimental.pallas import tpu as pltpu

_M = 4096
_N = 4096
_BM = 256
_R = _BM // 128


def _loss_body(A_ref, x_ref, b_ref, Iy_ref, il_ref, iu_ref, l_ref, u_ref,
               out_ref, acc_ref):
    i = pl.program_id(0)
    nb = pl.num_programs(0)

    @pl.when(i == 0)
    def _init():
        xv = x_ref[...]
        vv = (jnp.maximum(l_ref[...] - xv, 0.0) * il_ref[...]
              + jnp.maximum(xv - u_ref[...], 0.0) * iu_ref[...])
        bv = b_ref[...]
        acc_ref[0] = jnp.sum(vv * vv)
        acc_ref[1] = jnp.sum(bv * bv)
        acc_ref[2] = 0.0

    ax = jnp.sum(A_ref[...] * x_ref[...][None], axis=(1, 2))
    ax2 = ax.reshape(_R, 128)
    bb = b_ref[pl.ds(i * _R, _R), :]
    cv = bb - ax2
    cv = cv + jnp.maximum(-cv, 0.0) * Iy_ref[pl.ds(i * _R, _R), :]
    acc_ref[2] += jnp.sum(cv * cv)

    @pl.when(i == nb - 1)
    def _fin():
        part_2 = jnp.sqrt(acc_ref[0] + acc_ref[2])
        part_3 = 1.0 + jnp.sqrt(acc_ref[1])
        out_ref[0] = part_2 / part_3


def kernel(A, b, c, x, Iy, il, iu, l, u):
    del c
    A3 = A.reshape(_M, 32, 128)
    vecs = [v.reshape(32, 128) for v in (x, b, Iy, il, iu, l, u)]
    full = pl.BlockSpec((32, 128), lambda i: (0, 0))
    out = pl.pallas_call(
        _loss_body,
        grid=(_M // _BM,),
        in_specs=[pl.BlockSpec((_BM, 32, 128), lambda i: (i, 0, 0))]
        + [full] * 7,
        out_specs=pl.BlockSpec(memory_space=pltpu.SMEM),
        out_shape=jax.ShapeDtypeStruct((1,), jnp.float32),
        scratch_shapes=[pltpu.SMEM((3,), jnp.float32)],
    )(A3, *vecs)
    return out[0]


# speedup vs baseline: 2.9744x; 2.9744x over previous
import jax
import jax.numpy as jnp
from jax.experimental import pallas as pl
from jax.experimental.pallas import tpu as pltpu

_M = 4096
_N = 4096
_BG = 32
_BM = _BG * 8


def _loss_body(A_ref, xb_ref, b_ref, Iy_ref, x_ref, il_ref, iu_ref, l_ref,
               u_ref, out_ref, acc_ref):
    i = pl.program_id(0)
    nb = pl.num_programs(0)

    @pl.when(i == 0)
    def _init():
        xv = x_ref[...]
        vv = (jnp.maximum(l_ref[...] - xv, 0.0) * il_ref[...]
              + jnp.maximum(xv - u_ref[...], 0.0) * iu_ref[...])
        bv = b_ref[...]
        acc_ref[0] = jnp.sum(vv * vv)
        acc_ref[1] = jnp.sum(bv * bv)
        acc_ref[2] = 0.0

    ax = jnp.sum(A_ref[...] * xb_ref[...][None], axis=2)
    bb = b_ref[pl.ds(i * _BG, _BG), :]
    cv = bb - ax
    cv = cv + jnp.maximum(-cv, 0.0) * Iy_ref[pl.ds(i * _BG, _BG), :]
    acc_ref[2] += jnp.sum(cv * cv)

    @pl.when(i == nb - 1)
    def _fin():
        part_2 = jnp.sqrt(acc_ref[0] + acc_ref[2])
        part_3 = 1.0 + jnp.sqrt(acc_ref[1])
        out_ref[0] = part_2 / part_3


def kernel(A, b, c, x, Iy, il, iu, l, u):
    del c
    A3 = A.reshape(_M // 8, 8, _N)
    xb = jnp.broadcast_to(x.reshape(1, _N), (8, _N))
    b8 = b.reshape(_M // 8, 8)
    Iy8 = Iy.reshape(_M // 8, 8)
    small = [v.reshape(32, 128) for v in (x, il, iu, l, u)]
    full8 = pl.BlockSpec((_M // 8, 8), lambda i: (0, 0))
    full = pl.BlockSpec((32, 128), lambda i: (0, 0))
    out = pl.pallas_call(
        _loss_body,
        grid=(_M // _BM,),
        in_specs=[
            pl.BlockSpec((_BG, 8, _N), lambda i: (i, 0, 0)),
            pl.BlockSpec((8, _N), lambda i: (0, 0)),
            full8,
            full8,
            full,
            full,
            full,
            full,
            full,
        ],
        out_specs=pl.BlockSpec(memory_space=pltpu.SMEM),
        out_shape=jax.ShapeDtypeStruct((1,), jnp.float32),
        scratch_shapes=[pltpu.SMEM((3,), jnp.float32)],
    )(A3, xb, b8, Iy8, *small)
    return out[0]


# speedup vs baseline: 3.2804x; 1.1029x over previous
import jax
import jax.numpy as jnp
from jax.experimental import pallas as pl
from jax.experimental.pallas import tpu as pltpu

_M = 4096
_N = 4096
_BG = 64
_BM = _BG * 8


def _loss_body(A_ref, xb_ref, b_ref, Iy_ref, x_ref, il_ref, iu_ref, l_ref,
               u_ref, out_ref, acc_ref):
    i = pl.program_id(0)
    nb = pl.num_programs(0)

    @pl.when(i == 0)
    def _init():
        xv = x_ref[...]
        vv = (jnp.maximum(l_ref[...] - xv, 0.0) * il_ref[...]
              + jnp.maximum(xv - u_ref[...], 0.0) * iu_ref[...])
        bv = b_ref[...]
        acc_ref[0] = jnp.sum(vv * vv)
        acc_ref[1] = jnp.sum(bv * bv)
        acc_ref[2] = 0.0

    ax = jnp.sum(A_ref[...] * xb_ref[...][None], axis=2)
    bb = b_ref[pl.ds(i * _BG, _BG), :]
    cv = bb - ax
    cv = cv + jnp.maximum(-cv, 0.0) * Iy_ref[pl.ds(i * _BG, _BG), :]
    acc_ref[2] += jnp.sum(cv * cv)

    @pl.when(i == nb - 1)
    def _fin():
        part_2 = jnp.sqrt(acc_ref[0] + acc_ref[2])
        part_3 = 1.0 + jnp.sqrt(acc_ref[1])
        out_ref[0] = part_2 / part_3


def kernel(A, b, c, x, Iy, il, iu, l, u):
    del c
    A3 = A.reshape(_M // 8, 8, _N)
    xb = jnp.broadcast_to(x.reshape(1, _N), (8, _N))
    b8 = b.reshape(_M // 8, 8)
    Iy8 = Iy.reshape(_M // 8, 8)
    small = [v.reshape(32, 128) for v in (x, il, iu, l, u)]
    full8 = pl.BlockSpec((_M // 8, 8), lambda i: (0, 0))
    full = pl.BlockSpec((32, 128), lambda i: (0, 0))
    out = pl.pallas_call(
        _loss_body,
        grid=(_M // _BM,),
        in_specs=[
            pl.BlockSpec((_BG, 8, _N), lambda i: (i, 0, 0)),
            pl.BlockSpec((8, _N), lambda i: (0, 0)),
            full8,
            full8,
            full,
            full,
            full,
            full,
            full,
        ],
        out_specs=pl.BlockSpec(memory_space=pltpu.SMEM),
        out_shape=jax.ShapeDtypeStruct((1,), jnp.float32),
        scratch_shapes=[pltpu.SMEM((3,), jnp.float32)],
    )(A3, xb, b8, Iy8, *small)
    return out[0]
